# baseline (speedup 1.0000x reference)
import functools

import jax
import jax.numpy as jnp
from jax import lax
from jax.experimental import pallas as pl
from jax.experimental.pallas import tpu as pltpu
from jax.experimental.pallas import tpu_sc as plsc

B = 16384
D = 64
NC = 2
NS = 16
NW = NC * NS
L = 16

RANGE_W = 31232
NCHUNK = 16
CHUNK_W = 2048
CAP = 640
NGRP = CAP // L
EXTRA_W0 = 999424
PACK = NW * CAP

PHYS_TILED = False

_I16 = jnp.int32


def _lane():
    return lax.iota(jnp.int32, L)


def _scan_body(wi_hbm, wj_hbm, wt_hbm, wext_hbm,
               packed_i, packed_j, slots_i, slots_j,
               piece_a, piece_b, tw_i, ts_i, tw_j, ts_j,
               bw_i, bs_i, bw_j, bs_j, rows_i, rows_j,
               chunk_a, chunk_b, co_smem, sem_a, sem_b):
    wid = lax.axis_index("s") * NC + lax.axis_index("c")
    lo = wid * RANGE_W
    hi = jnp.where(wid == NW - 1, 1000000, lo + RANGE_W)
    lane = _lane()

    for g in range(NGRP):
        slots_sl = pl.ds(g * L, L)
        dump = B + wid * CAP + g * L + lane
        bs_i[slots_sl] = dump
        bs_j[slots_sl] = dump

    def run_stream(idx_hbm, tw, ts):
        cp0 = pltpu.async_copy(idx_hbm.at[pl.ds(0, 1024)], piece_a, sem_a)

        def do_piece(p, pbuf, n, cp, cpn):
            cp.wait()

            def vbody(v, n):
                w = pbuf[pl.ds(v * L, L)]
                slot = p * 1024 + v * L + lane
                m = (w >= lo) & (w < hi)
                nc = jnp.minimum(n, CAP - L)
                plsc.store_compressed(tw.at[pl.ds(nc, L)], w, mask=m)
                plsc.store_compressed(ts.at[pl.ds(nc, L)], slot, mask=m)
                return n + jnp.sum(m.astype(jnp.int32))

            n = lax.fori_loop(0, 64, vbody, n)
            return n, cpn

        n = jnp.int32(0)
        cp = cp0
        for p in range(16):
            if p < 15:
                nbuf = piece_b if p % 2 == 0 else piece_a
                cpn = pltpu.async_copy(
                    idx_hbm.at[pl.ds((p + 1) * 1024, 1024)], nbuf,
                    sem_b if p % 2 == 0 else sem_a)
            else:
                cpn = None
            pbuf = piece_a if p % 2 == 0 else piece_b
            n, cp = do_piece(p, pbuf, n, cp, cpn)
        return jnp.minimum(n, CAP)

    n_i = run_stream(wi_hbm, tw_i, ts_i)
    n_j = run_stream(wj_hbm, tw_j, ts_j)

    def sort_stream(tw, ts, bw, bs, n, co_base):
        def chunk_pass(off, c):
            co_smem[co_base + c] = off

            def inner(g, off):
                sl = pl.ds(g * L, L)
                w = tw[sl]
                s = ts[sl]
                valid = (g * L + lane) < n
                ch = lax.shift_right_logical(w - lo, 11)
                m = valid & (ch == c)
                oc = jnp.minimum(off, CAP - L)
                plsc.store_compressed(bw.at[pl.ds(oc, L)], w, mask=m)
                plsc.store_compressed(bs.at[pl.ds(oc, L)], s, mask=m)
                return off + jnp.sum(m.astype(jnp.int32))

            ngrp = lax.shift_right_logical(n + (L - 1), 4)
            return lax.fori_loop(0, ngrp, inner, off)

        off = jnp.int32(0)
        for c in range(NCHUNK):
            off = chunk_pass(off, c)
        co_smem[co_base + NCHUNK] = off

    sort_stream(tw_i, ts_i, bw_i, bs_i, n_i, 0)
    sort_stream(tw_j, ts_j, bw_j, bs_j, n_j, NCHUNK + 1)

    def extract(buf, tr, c_id, w0):
        def one_stream(bw, bs, rows, co_base):
            s0 = co_smem[co_base + c_id]
            s1 = jnp.minimum(co_smem[co_base + c_id + 1], CAP)

            def egroup(g, carry):
                pos = s0 + g * L
                w16 = bw[pl.ds(pos, L)]
                loc = w16 - w0
                m = (pos + lane) < s1
                if PHYS_TILED:
                    tc = lax.shift_right_logical(loc, 7)
                    phys = tc * 1024 + jnp.bitwise_and(loc, 127)
                else:
                    phys = loc
                for fi in range(8):
                    if PHYS_TILED:
                        p2 = phys + fi * 128
                        i0 = lax.shift_right_logical(p2, 11)
                        i1 = jnp.bitwise_and(p2, 2047)
                    else:
                        i0 = jnp.full((L,), fi, jnp.int32)
                        i1 = phys
                    v = plsc.load_gather(buf, [i0, i1], mask=m)
                    plsc.store_scatter(
                        rows, [(pos + lane) * D + tr * 8 + fi], v, mask=m)
                return carry

            ngrp = lax.shift_right_logical(s1 - s0 + (L - 1), 4)
            lax.fori_loop(0, ngrp, egroup, 0)

        one_stream(bw_i, bs_i, rows_i, 0)
        one_stream(bw_j, bs_j, rows_j, NCHUNK + 1)

    def issue(step, buf, sem):
        tr = lax.div(step, 15)
        c = lax.rem(step, 15)
        r0 = pl.multiple_of(tr * 8, 8)
        w0 = pl.multiple_of(lo + c * CHUNK_W, 128)
        return pltpu.async_copy(
            wt_hbm.at[pl.ds(r0, 8), pl.ds(w0, CHUNK_W)], buf, sem)

    cp_a = issue(jnp.int32(0), chunk_a, sem_a)
    cp_b = issue(jnp.int32(1), chunk_b, sem_b)

    def two_steps(k, carry):
        step = k * 2

        def body_one(step, buf, sem):
            pltpu.make_async_copy(
                wt_hbm.at[pl.ds(0, 8), pl.ds(0, CHUNK_W)], buf, sem).wait()
            tr = lax.div(step, 15)
            c = lax.rem(step, 15)
            extract(buf, tr, c, lo + c * CHUNK_W)

            @pl.when(step + 2 < 120)
            def _():
                tr2 = lax.div(step + 2, 15)
                c2 = lax.rem(step + 2, 15)
                r0 = pl.multiple_of(tr2 * 8, 8)
                w0 = pl.multiple_of(lo + c2 * CHUNK_W, 128)
                pltpu.make_async_copy(
                    wt_hbm.at[pl.ds(r0, 8), pl.ds(w0, CHUNK_W)], buf,
                    sem).start()

        body_one(step, chunk_a, sem_a)
        body_one(step + 1, chunk_b, sem_b)
        return carry

    lax.fori_loop(0, 60, two_steps, 0)

    for tr in range(8):
        w0 = pl.multiple_of(lo + 15 * CHUNK_W, 128)
        pltpu.sync_copy(
            wt_hbm.at[pl.ds(tr * 8, 8), pl.ds(w0, 512)],
            chunk_a.at[pl.ds(0, 8), pl.ds(0, 512)])

        @pl.when(wid == NW - 1)
        def _():
            pltpu.sync_copy(
                wext_hbm.at[pl.ds(tr * 8, 8), pl.ds(0, 640)],
                chunk_a.at[pl.ds(0, 8), pl.ds(512, 640)])

        extract(chunk_a, tr, 15, lo + 15 * CHUNK_W)

    pltpu.sync_copy(rows_i, packed_i.at[pl.ds(wid * CAP * D, CAP * D)])
    pltpu.sync_copy(rows_j, packed_j.at[pl.ds(wid * CAP * D, CAP * D)])
    pltpu.sync_copy(bs_i, slots_i.at[pl.ds(wid * CAP, CAP)])
    pltpu.sync_copy(bs_j, slots_j.at[pl.ds(wid * CAP, CAP)])


def _board_body(packed_i, packed_j, slots_i, slots_j,
                board_i, board_j, rowsv, slotv, sem):
    wid = lax.axis_index("s") * NC + lax.axis_index("c")

    def one(packed, slots, board):
        pltpu.sync_copy(packed.at[pl.ds(wid * CAP, CAP), pl.ds(0, D)], rowsv)
        pltpu.sync_copy(slots.at[wid], slotv)
        cps = []
        for j in range(CAP // 128):
            cps.append(pltpu.async_copy(
                rowsv.at[pl.ds(j * 128, 128), pl.ds(0, D)],
                board.at[slotv.at[j]], sem))
        for cp in cps:
            cp.wait()

    one(packed_i, slots_i, board_i)
    one(packed_j, slots_j, board_j)


def _dot_body(board_i, board_j, wi_hbm, wj_hbm, b_hbm, out_hbm,
              vi, vj, idx_i, idx_j, bi, bj, out_v, sem):
    wid = lax.axis_index("s") * NC + lax.axis_index("c")
    bpw = B // NW
    base = wid * bpw

    cps = [
        pltpu.async_copy(board_i.at[pl.ds(base, bpw), pl.ds(0, D)], vi, sem),
        pltpu.async_copy(board_j.at[pl.ds(base, bpw), pl.ds(0, D)], vj, sem),
    ]
    pltpu.sync_copy(wi_hbm.at[pl.ds(base, bpw)], idx_i)
    pltpu.sync_copy(wj_hbm.at[pl.ds(base, bpw)], idx_j)
    bias_cps = []
    for c in range(bpw // 128):
        sl = pl.ds(c * 128, 128)
        bias_cps.append(pltpu.async_copy(b_hbm.at[idx_i.at[sl]], bi.at[sl], sem))
        bias_cps.append(pltpu.async_copy(b_hbm.at[idx_j.at[sl]], bj.at[sl], sem))
    for cp in cps:
        cp.wait()

    lane = _lane()

    def group_body(g, carry):
        rows = g * L + lane
        col = jnp.full((L,), 0, jnp.int32)
        acc = plsc.load_gather(vi, [rows, col]) * plsc.load_gather(vj, [rows, col])
        for k in range(1, D):
            col = jnp.full((L,), k, jnp.int32)
            a = plsc.load_gather(vi, [rows, col])
            bvec = plsc.load_gather(vj, [rows, col])
            acc = acc + a * bvec
        out_v[pl.ds(g * L, L)] = acc
        return carry

    lax.fori_loop(0, bpw // L, group_body, 0)
    for cp in bias_cps:
        cp.wait()

    def bias_body(g, carry):
        sl = pl.ds(g * L, L)
        out_v[sl] = out_v[sl] + bi[sl] + bj[sl]
        return carry

    lax.fori_loop(0, bpw // L, bias_body, 0)
    pltpu.sync_copy(out_v, out_hbm.at[pl.ds(base, bpw)])


@jax.jit
def _glove(w_i, w_j, W, b):
    mesh = plsc.VectorSubcoreMesh(core_axis_name="c", subcore_axis_name="s")
    wt = W.T
    wext = jnp.pad(wt[:, EXTRA_W0:], ((0, 0), (0, 640 - (1000000 - EXTRA_W0))))

    scan = functools.partial(
        pl.kernel,
        mesh=mesh,
        compiler_params=pltpu.CompilerParams(
            needs_layout_passes=False, use_tc_tiling_on_sc=True),
        out_type=(
            pltpu.HBM((PACK * D,), jnp.float32),
            pltpu.HBM((PACK * D,), jnp.float32),
            pltpu.HBM((PACK,), jnp.int32),
            pltpu.HBM((PACK,), jnp.int32),
        ),
        scratch_types=[
            pltpu.VMEM((1024,), jnp.int32),
            pltpu.VMEM((1024,), jnp.int32),
            pltpu.VMEM((CAP,), jnp.int32),
            pltpu.VMEM((CAP,), jnp.int32),
            pltpu.VMEM((CAP,), jnp.int32),
            pltpu.VMEM((CAP,), jnp.int32),
            pltpu.VMEM((CAP,), jnp.int32),
            pltpu.VMEM((CAP,), jnp.int32),
            pltpu.VMEM((CAP,), jnp.int32),
            pltpu.VMEM((CAP,), jnp.int32),
            pltpu.VMEM((CAP * D,), jnp.float32),
            pltpu.VMEM((CAP * D,), jnp.float32),
            pltpu.VMEM((8, CHUNK_W), jnp.float32),
            pltpu.VMEM((8, CHUNK_W), jnp.float32),
            pltpu.SMEM((2 * (NCHUNK + 1),), jnp.int32),
            pltpu.SemaphoreType.DMA,
            pltpu.SemaphoreType.DMA,
        ],
    )(_scan_body)
    packed_i, packed_j, slots_i, slots_j = scan(w_i, w_j, wt, wext)

    board = functools.partial(
        pl.kernel,
        mesh=mesh,
        compiler_params=pltpu.CompilerParams(
            needs_layout_passes=False, use_tc_tiling_on_sc=False),
        out_type=(
            pltpu.HBM((B + PACK, D), jnp.float32),
            pltpu.HBM((B + PACK, D), jnp.float32),
        ),
        scratch_types=[
            pltpu.VMEM((CAP, D), jnp.float32),
            pltpu.VMEM((CAP // 128, 128), jnp.int32),
            pltpu.SemaphoreType.DMA,
        ],
    )(_board_body)
    board_i, board_j = board(
        packed_i.reshape(PACK, D), packed_j.reshape(PACK, D),
        slots_i.reshape(NW, CAP // 128, 128),
        slots_j.reshape(NW, CAP // 128, 128))

    dot = functools.partial(
        pl.kernel,
        mesh=mesh,
        compiler_params=pltpu.CompilerParams(
            needs_layout_passes=False, use_tc_tiling_on_sc=False),
        out_type=jax.ShapeDtypeStruct((B,), jnp.float32),
        scratch_types=[
            pltpu.VMEM((B // NW, D), jnp.float32),
            pltpu.VMEM((B // NW, D), jnp.float32),
            pltpu.VMEM((B // NW,), jnp.int32),
            pltpu.VMEM((B // NW,), jnp.int32),
            pltpu.VMEM((B // NW,), jnp.float32),
            pltpu.VMEM((B // NW,), jnp.float32),
            pltpu.VMEM((B // NW,), jnp.float32),
            pltpu.SemaphoreType.DMA,
        ],
    )(_dot_body)
    return dot(board_i, board_j, w_i, w_j, b)


def kernel(w_i, w_j, W, b):
    return _glove(w_i.astype(jnp.int32), w_j.astype(jnp.int32), W, b)


# speedup vs baseline: 1.1197x; 1.1197x over previous
import functools

import jax
import jax.numpy as jnp
from jax import lax
from jax.experimental import pallas as pl
from jax.experimental.pallas import tpu as pltpu
from jax.experimental.pallas import tpu_sc as plsc

B = 16384
D = 64
NC = 2
NS = 16
NW = NC * NS
L = 16

RANGE_W = 31232
NCHUNK = 16
CHUNK_W = 2048
CAP = 640
NGRP = CAP // L
EXTRA_W0 = 999424
PACK = NW * CAP

PHYS_TILED = False

_I16 = jnp.int32


def _lane():
    return lax.iota(jnp.int32, L)


def _scan_body(wi_hbm, wj_hbm, wt_hbm, wext_hbm,
               packed_i, packed_j, slots_i, slots_j,
               piece_a, piece_b, tw_i, ts_i, tw_j, ts_j,
               bw_i, bs_i, bw_j, bs_j, rows_i, rows_j,
               chunk_a, chunk_b, co_smem, sem_a, sem_b):
    wid = lax.axis_index("s") * NC + lax.axis_index("c")
    lo = wid * RANGE_W
    hi = jnp.where(wid == NW - 1, 1000000, lo + RANGE_W)
    lane = _lane()

    for g in range(NGRP):
        slots_sl = pl.ds(g * L, L)
        dump = B + wid * CAP + g * L + lane
        bs_i[slots_sl] = dump
        bs_j[slots_sl] = dump

    def run_stream(idx_hbm, tw, ts):
        cp0 = pltpu.async_copy(idx_hbm.at[pl.ds(0, 1024)], piece_a, sem_a)

        def do_piece(p, pbuf, n, cp, cpn):
            cp.wait()

            def vbody(v, n):
                w = pbuf[pl.ds(v * L, L)]
                slot = p * 1024 + v * L + lane
                m = (w >= lo) & (w < hi)
                nc = jnp.minimum(n, CAP - L)
                plsc.store_compressed(tw.at[pl.ds(nc, L)], w, mask=m)
                plsc.store_compressed(ts.at[pl.ds(nc, L)], slot, mask=m)
                return n + jnp.sum(m.astype(jnp.int32))

            n = lax.fori_loop(0, 64, vbody, n)
            return n, cpn

        n = jnp.int32(0)
        cp = cp0
        for p in range(16):
            if p < 15:
                nbuf = piece_b if p % 2 == 0 else piece_a
                cpn = pltpu.async_copy(
                    idx_hbm.at[pl.ds((p + 1) * 1024, 1024)], nbuf,
                    sem_b if p % 2 == 0 else sem_a)
            else:
                cpn = None
            pbuf = piece_a if p % 2 == 0 else piece_b
            n, cp = do_piece(p, pbuf, n, cp, cpn)
        return jnp.minimum(n, CAP)

    n_i = run_stream(wi_hbm, tw_i, ts_i)
    n_j = run_stream(wj_hbm, tw_j, ts_j)

    def sort_stream(tw, ts, bw, bs, n, co_base):
        def chunk_pass(off, c):
            co_smem[co_base + c] = off

            def inner(g, off):
                sl = pl.ds(g * L, L)
                w = tw[sl]
                s = ts[sl]
                valid = (g * L + lane) < n
                ch = lax.shift_right_logical(w - lo, 11)
                m = valid & (ch == c)
                oc = jnp.minimum(off, CAP - L)
                plsc.store_compressed(bw.at[pl.ds(oc, L)], w, mask=m)
                plsc.store_compressed(bs.at[pl.ds(oc, L)], s, mask=m)
                return off + jnp.sum(m.astype(jnp.int32))

            ngrp = lax.shift_right_logical(n + (L - 1), 4)
            return lax.fori_loop(0, ngrp, inner, off)

        off = jnp.int32(0)
        for c in range(NCHUNK):
            off = chunk_pass(off, c)
        co_smem[co_base + NCHUNK] = off

    sort_stream(tw_i, ts_i, bw_i, bs_i, n_i, 0)
    sort_stream(tw_j, ts_j, bw_j, bs_j, n_j, NCHUNK + 1)

    def extract(buf, tr, c_id, w0):
        def one_stream(bw, bs, rows, co_base):
            s0 = co_smem[co_base + c_id]
            s1 = jnp.minimum(co_smem[co_base + c_id + 1], CAP)

            def egroup(g, carry):
                pos = s0 + g * L
                w16 = bw[pl.ds(pos, L)]
                loc = w16 - w0
                m = (pos + lane) < s1
                if PHYS_TILED:
                    tc = lax.shift_right_logical(loc, 7)
                    phys = tc * 1024 + jnp.bitwise_and(loc, 127)
                else:
                    phys = loc
                for fi in range(8):
                    if PHYS_TILED:
                        p2 = phys + fi * 128
                        i0 = lax.shift_right_logical(p2, 11)
                        i1 = jnp.bitwise_and(p2, 2047)
                    else:
                        i0 = jnp.full((L,), fi, jnp.int32)
                        i1 = phys
                    v = plsc.load_gather(buf, [i0, i1], mask=m)
                    plsc.store_scatter(
                        rows, [(pos + lane) * D + tr * 8 + fi], v, mask=m)
                return carry

            ngrp = lax.shift_right_logical(s1 - s0 + (L - 1), 4)
            lax.fori_loop(0, ngrp, egroup, 0)

        one_stream(bw_i, bs_i, rows_i, 0)
        one_stream(bw_j, bs_j, rows_j, NCHUNK + 1)

    def issue(step, buf, sem):
        tr = lax.div(step, 15)
        c = lax.rem(step, 15)
        r0 = pl.multiple_of(tr * 8, 8)
        w0 = pl.multiple_of(lo + c * CHUNK_W, 128)
        return pltpu.async_copy(
            wt_hbm.at[pl.ds(r0, 8), pl.ds(w0, CHUNK_W)], buf, sem)

    cp_a = issue(jnp.int32(0), chunk_a, sem_a)
    cp_b = issue(jnp.int32(1), chunk_b, sem_b)

    def two_steps(k, carry):
        step = k * 2

        def body_one(step, buf, sem):
            pltpu.make_async_copy(
                wt_hbm.at[pl.ds(0, 8), pl.ds(0, CHUNK_W)], buf, sem).wait()
            tr = lax.div(step, 15)
            c = lax.rem(step, 15)
            extract(buf, tr, c, lo + c * CHUNK_W)

            @pl.when(step + 2 < 120)
            def _():
                tr2 = lax.div(step + 2, 15)
                c2 = lax.rem(step + 2, 15)
                r0 = pl.multiple_of(tr2 * 8, 8)
                w0 = pl.multiple_of(lo + c2 * CHUNK_W, 128)
                pltpu.make_async_copy(
                    wt_hbm.at[pl.ds(r0, 8), pl.ds(w0, CHUNK_W)], buf,
                    sem).start()

        body_one(step, chunk_a, sem_a)
        body_one(step + 1, chunk_b, sem_b)
        return carry

    lax.fori_loop(0, 60, two_steps, 0)

    for tr in range(8):
        w0 = pl.multiple_of(lo + 15 * CHUNK_W, 128)
        pltpu.sync_copy(
            wt_hbm.at[pl.ds(tr * 8, 8), pl.ds(w0, 512)],
            chunk_a.at[pl.ds(0, 8), pl.ds(0, 512)])

        @pl.when(wid == NW - 1)
        def _():
            pltpu.sync_copy(
                wext_hbm.at[pl.ds(tr * 8, 8), pl.ds(0, 640)],
                chunk_a.at[pl.ds(0, 8), pl.ds(512, 640)])

        extract(chunk_a, tr, 15, lo + 15 * CHUNK_W)

    pltpu.sync_copy(rows_i, packed_i.at[pl.ds(wid * CAP * D, CAP * D)])
    pltpu.sync_copy(rows_j, packed_j.at[pl.ds(wid * CAP * D, CAP * D)])
    pltpu.sync_copy(bs_i, slots_i.at[pl.ds(wid * CAP, CAP)])
    pltpu.sync_copy(bs_j, slots_j.at[pl.ds(wid * CAP, CAP)])


def _board_body(packed_i, packed_j, slots_i, slots_j,
                board_i, board_j, rowsv, slotv, sem):
    wid = lax.axis_index("s") * NC + lax.axis_index("c")

    def one(packed, slots, board):
        pltpu.sync_copy(packed.at[pl.ds(wid * CAP, CAP), pl.ds(0, D)], rowsv)
        pltpu.sync_copy(slots.at[wid], slotv)
        cps = []
        for j in range(CAP // 128):
            cps.append(pltpu.async_copy(
                rowsv.at[pl.ds(j * 128, 128), pl.ds(0, D)],
                board.at[slotv.at[j]], sem))
        for cp in cps:
            cp.wait()

    one(packed_i, slots_i, board_i)
    one(packed_j, slots_j, board_j)


def _dot_body(board_i, board_j, wi_hbm, wj_hbm, b_hbm, out_hbm,
              vi, vj, idx_i, idx_j, bi, bj, out_v, sem):
    wid = lax.axis_index("s") * NC + lax.axis_index("c")
    bpw = B // NW
    base = wid * bpw

    cps = [
        pltpu.async_copy(board_i.at[pl.ds(base, bpw), pl.ds(0, D)], vi, sem),
        pltpu.async_copy(board_j.at[pl.ds(base, bpw), pl.ds(0, D)], vj, sem),
    ]
    pltpu.sync_copy(wi_hbm.at[pl.ds(base, bpw)], idx_i)
    pltpu.sync_copy(wj_hbm.at[pl.ds(base, bpw)], idx_j)
    bias_cps = []
    for c in range(bpw // 128):
        sl = pl.ds(c * 128, 128)
        bias_cps.append(pltpu.async_copy(b_hbm.at[idx_i.at[sl]], bi.at[sl], sem))
        bias_cps.append(pltpu.async_copy(b_hbm.at[idx_j.at[sl]], bj.at[sl], sem))
    for cp in cps:
        cp.wait()

    lane = _lane()

    def group_body(g, carry):
        rows = g * L + lane
        col = jnp.bitwise_and(lane, D - 1)
        acc = plsc.load_gather(vi, [rows, col]) * plsc.load_gather(vj, [rows, col])
        for k in range(1, D):
            col = jnp.bitwise_and(lane + k, D - 1)
            a = plsc.load_gather(vi, [rows, col])
            bvec = plsc.load_gather(vj, [rows, col])
            acc = acc + a * bvec
        out_v[pl.ds(g * L, L)] = acc
        return carry

    lax.fori_loop(0, bpw // L, group_body, 0)
    for cp in bias_cps:
        cp.wait()

    def bias_body(g, carry):
        sl = pl.ds(g * L, L)
        out_v[sl] = out_v[sl] + bi[sl] + bj[sl]
        return carry

    lax.fori_loop(0, bpw // L, bias_body, 0)
    pltpu.sync_copy(out_v, out_hbm.at[pl.ds(base, bpw)])


@jax.jit
def _glove(w_i, w_j, W, b):
    mesh = plsc.VectorSubcoreMesh(core_axis_name="c", subcore_axis_name="s")
    wt = W.T
    wext = jnp.pad(wt[:, EXTRA_W0:], ((0, 0), (0, 640 - (1000000 - EXTRA_W0))))

    scan = functools.partial(
        pl.kernel,
        mesh=mesh,
        compiler_params=pltpu.CompilerParams(
            needs_layout_passes=False, use_tc_tiling_on_sc=True),
        out_type=(
            pltpu.HBM((PACK * D,), jnp.float32),
            pltpu.HBM((PACK * D,), jnp.float32),
            pltpu.HBM((PACK,), jnp.int32),
            pltpu.HBM((PACK,), jnp.int32),
        ),
        scratch_types=[
            pltpu.VMEM((1024,), jnp.int32),
            pltpu.VMEM((1024,), jnp.int32),
            pltpu.VMEM((CAP,), jnp.int32),
            pltpu.VMEM((CAP,), jnp.int32),
            pltpu.VMEM((CAP,), jnp.int32),
            pltpu.VMEM((CAP,), jnp.int32),
            pltpu.VMEM((CAP,), jnp.int32),
            pltpu.VMEM((CAP,), jnp.int32),
            pltpu.VMEM((CAP,), jnp.int32),
            pltpu.VMEM((CAP,), jnp.int32),
            pltpu.VMEM((CAP * D,), jnp.float32),
            pltpu.VMEM((CAP * D,), jnp.float32),
            pltpu.VMEM((8, CHUNK_W), jnp.float32),
            pltpu.VMEM((8, CHUNK_W), jnp.float32),
            pltpu.SMEM((2 * (NCHUNK + 1),), jnp.int32),
            pltpu.SemaphoreType.DMA,
            pltpu.SemaphoreType.DMA,
        ],
    )(_scan_body)
    packed_i, packed_j, slots_i, slots_j = scan(w_i, w_j, wt, wext)

    board = functools.partial(
        pl.kernel,
        mesh=mesh,
        compiler_params=pltpu.CompilerParams(
            needs_layout_passes=False, use_tc_tiling_on_sc=False),
        out_type=(
            pltpu.HBM((B + PACK, D), jnp.float32),
            pltpu.HBM((B + PACK, D), jnp.float32),
        ),
        scratch_types=[
            pltpu.VMEM((CAP, D), jnp.float32),
            pltpu.VMEM((CAP // 128, 128), jnp.int32),
            pltpu.SemaphoreType.DMA,
        ],
    )(_board_body)
    board_i, board_j = board(
        packed_i.reshape(PACK, D), packed_j.reshape(PACK, D),
        slots_i.reshape(NW, CAP // 128, 128),
        slots_j.reshape(NW, CAP // 128, 128))

    dot = functools.partial(
        pl.kernel,
        mesh=mesh,
        compiler_params=pltpu.CompilerParams(
            needs_layout_passes=False, use_tc_tiling_on_sc=False),
        out_type=jax.ShapeDtypeStruct((B,), jnp.float32),
        scratch_types=[
            pltpu.VMEM((B // NW, D), jnp.float32),
            pltpu.VMEM((B // NW, D), jnp.float32),
            pltpu.VMEM((B // NW,), jnp.int32),
            pltpu.VMEM((B // NW,), jnp.int32),
            pltpu.VMEM((B // NW,), jnp.float32),
            pltpu.VMEM((B // NW,), jnp.float32),
            pltpu.VMEM((B // NW,), jnp.float32),
            pltpu.SemaphoreType.DMA,
        ],
    )(_dot_body)
    return dot(board_i, board_j, w_i, w_j, b)


def kernel(w_i, w_j, W, b):
    return _glove(w_i.astype(jnp.int32), w_j.astype(jnp.int32), W, b)


# speedup vs baseline: 1.1622x; 1.0380x over previous
import functools

import jax
import jax.numpy as jnp
from jax import lax
from jax.experimental import pallas as pl
from jax.experimental.pallas import tpu as pltpu
from jax.experimental.pallas import tpu_sc as plsc

B = 16384
D = 64
NC = 2
NS = 16
NW = NC * NS
L = 16

RANGE_W = 31232
NCHUNK = 16
CHUNK_W = 2048
CAP = 640
NGRP = CAP // L
EXTRA_W0 = 999424
PACK = NW * CAP

PHYS_TILED = False

_I16 = jnp.int32


def _lane():
    return lax.iota(jnp.int32, L)


def _scan_body(wi_hbm, wj_hbm, wt_hbm, wext_hbm,
               packed_i, packed_j, slots_i, slots_j,
               piece_a, piece_b, tw_i, ts_i, tw_j, ts_j,
               bw_i, bs_i, bw_j, bs_j, rows_i, rows_j,
               chunk_a, chunk_b, co_smem, sem_a, sem_b):
    wid = lax.axis_index("s") * NC + lax.axis_index("c")
    lo = wid * RANGE_W
    hi = jnp.where(wid == NW - 1, 1000000, lo + RANGE_W)
    lane = _lane()

    for g in range(NGRP):
        slots_sl = pl.ds(g * L, L)
        dump = B + wid * CAP + g * L + lane
        bs_i[slots_sl] = dump
        bs_j[slots_sl] = dump

    def run_stream(idx_hbm, tw, ts):
        cp0 = pltpu.async_copy(idx_hbm.at[pl.ds(0, 1024)], piece_a, sem_a)

        def do_piece(p, pbuf, n, cp, cpn):
            cp.wait()

            def vbody(v, n):
                w = pbuf[pl.ds(v * L, L)]
                slot = p * 1024 + v * L + lane
                m = (w >= lo) & (w < hi)
                nc = jnp.minimum(n, CAP - L)
                plsc.store_compressed(tw.at[pl.ds(nc, L)], w, mask=m)
                plsc.store_compressed(ts.at[pl.ds(nc, L)], slot, mask=m)
                return n + plsc.all_reduce_population_count(m)[0]

            n = lax.fori_loop(0, 64, vbody, n)
            return n, cpn

        n = jnp.int32(0)
        cp = cp0
        for p in range(16):
            if p < 15:
                nbuf = piece_b if p % 2 == 0 else piece_a
                cpn = pltpu.async_copy(
                    idx_hbm.at[pl.ds((p + 1) * 1024, 1024)], nbuf,
                    sem_b if p % 2 == 0 else sem_a)
            else:
                cpn = None
            pbuf = piece_a if p % 2 == 0 else piece_b
            n, cp = do_piece(p, pbuf, n, cp, cpn)
        return jnp.minimum(n, CAP)

    n_i = run_stream(wi_hbm, tw_i, ts_i)
    n_j = run_stream(wj_hbm, tw_j, ts_j)

    def sort_stream(tw, ts, bw, bs, n, co_base):
        def chunk_pass(off, c):
            co_smem[co_base + c] = off

            def inner(g, off):
                sl = pl.ds(g * L, L)
                w = tw[sl]
                s = ts[sl]
                valid = (g * L + lane) < n
                ch = lax.shift_right_logical(w - lo, 11)
                m = valid & (ch == c)
                oc = jnp.minimum(off, CAP - L)
                plsc.store_compressed(bw.at[pl.ds(oc, L)], w, mask=m)
                plsc.store_compressed(bs.at[pl.ds(oc, L)], s, mask=m)
                return off + plsc.all_reduce_population_count(m)[0]

            ngrp = lax.shift_right_logical(n + (L - 1), 4)
            return lax.fori_loop(0, ngrp, inner, off)

        off = jnp.int32(0)
        for c in range(NCHUNK):
            off = chunk_pass(off, c)
        co_smem[co_base + NCHUNK] = off

    sort_stream(tw_i, ts_i, bw_i, bs_i, n_i, 0)
    sort_stream(tw_j, ts_j, bw_j, bs_j, n_j, NCHUNK + 1)

    def extract(buf, tr, c_id, w0):
        def one_stream(bw, bs, rows, co_base):
            s0 = co_smem[co_base + c_id]
            s1 = jnp.minimum(co_smem[co_base + c_id + 1], CAP)

            def egroup(g, carry):
                pos = s0 + g * L
                w16 = bw[pl.ds(pos, L)]
                loc = w16 - w0
                m = (pos + lane) < s1
                if PHYS_TILED:
                    tc = lax.shift_right_logical(loc, 7)
                    phys = tc * 1024 + jnp.bitwise_and(loc, 127)
                else:
                    phys = loc
                for fi in range(8):
                    if PHYS_TILED:
                        p2 = phys + fi * 128
                        i0 = lax.shift_right_logical(p2, 11)
                        i1 = jnp.bitwise_and(p2, 2047)
                    else:
                        i0 = jnp.full((L,), fi, jnp.int32)
                        i1 = phys
                    v = plsc.load_gather(buf, [i0, i1], mask=m)
                    plsc.store_scatter(
                        rows, [(pos + lane) * D + tr * 8 + fi], v, mask=m)
                return carry

            ngrp = lax.shift_right_logical(s1 - s0 + (L - 1), 4)
            lax.fori_loop(0, ngrp, egroup, 0)

        one_stream(bw_i, bs_i, rows_i, 0)
        one_stream(bw_j, bs_j, rows_j, NCHUNK + 1)

    def issue(step, buf, sem):
        tr = lax.div(step, 15)
        c = lax.rem(step, 15)
        r0 = pl.multiple_of(tr * 8, 8)
        w0 = pl.multiple_of(lo + c * CHUNK_W, 128)
        return pltpu.async_copy(
            wt_hbm.at[pl.ds(r0, 8), pl.ds(w0, CHUNK_W)], buf, sem)

    cp_a = issue(jnp.int32(0), chunk_a, sem_a)
    cp_b = issue(jnp.int32(1), chunk_b, sem_b)

    def two_steps(k, carry):
        step = k * 2

        def body_one(step, buf, sem):
            pltpu.make_async_copy(
                wt_hbm.at[pl.ds(0, 8), pl.ds(0, CHUNK_W)], buf, sem).wait()
            tr = lax.div(step, 15)
            c = lax.rem(step, 15)
            extract(buf, tr, c, lo + c * CHUNK_W)

            @pl.when(step + 2 < 120)
            def _():
                tr2 = lax.div(step + 2, 15)
                c2 = lax.rem(step + 2, 15)
                r0 = pl.multiple_of(tr2 * 8, 8)
                w0 = pl.multiple_of(lo + c2 * CHUNK_W, 128)
                pltpu.make_async_copy(
                    wt_hbm.at[pl.ds(r0, 8), pl.ds(w0, CHUNK_W)], buf,
                    sem).start()

        body_one(step, chunk_a, sem_a)
        body_one(step + 1, chunk_b, sem_b)
        return carry

    lax.fori_loop(0, 60, two_steps, 0)

    w0_15 = pl.multiple_of(lo + 15 * CHUNK_W, 128)

    def issue15(tr, buf, sem):
        return pltpu.async_copy(
            wt_hbm.at[pl.ds(tr * 8, 8), pl.ds(w0_15, 512)],
            buf.at[pl.ds(0, 8), pl.ds(0, 512)], sem)

    cp15 = issue15(0, chunk_a, sem_a)
    for tr in range(8):
        buf = chunk_a if tr % 2 == 0 else chunk_b
        nxt = None
        if tr < 7:
            nxt = issue15(tr + 1,
                          chunk_b if tr % 2 == 0 else chunk_a,
                          sem_b if tr % 2 == 0 else sem_a)
        cp15.wait()

        @pl.when(wid == NW - 1)
        def _(tr=tr, buf=buf):
            pltpu.sync_copy(
                wext_hbm.at[pl.ds(tr * 8, 8), pl.ds(0, 640)],
                buf.at[pl.ds(0, 8), pl.ds(512, 640)])

        extract(buf, tr, 15, lo + 15 * CHUNK_W)
        cp15 = nxt

    pltpu.sync_copy(rows_i, packed_i.at[pl.ds(wid * CAP * D, CAP * D)])
    pltpu.sync_copy(rows_j, packed_j.at[pl.ds(wid * CAP * D, CAP * D)])
    pltpu.sync_copy(bs_i, slots_i.at[pl.ds(wid * CAP, CAP)])
    pltpu.sync_copy(bs_j, slots_j.at[pl.ds(wid * CAP, CAP)])


def _board_body(packed_i, packed_j, slots_i, slots_j,
                board_i, board_j, rowsv, slotv, sem):
    wid = lax.axis_index("s") * NC + lax.axis_index("c")

    def one(packed, slots, board):
        pltpu.sync_copy(packed.at[pl.ds(wid * CAP, CAP), pl.ds(0, D)], rowsv)
        pltpu.sync_copy(slots.at[wid], slotv)
        cps = []
        for j in range(CAP // 128):
            cps.append(pltpu.async_copy(
                rowsv.at[pl.ds(j * 128, 128), pl.ds(0, D)],
                board.at[slotv.at[j]], sem))
        for cp in cps:
            cp.wait()

    one(packed_i, slots_i, board_i)
    one(packed_j, slots_j, board_j)


def _dot_body(board_i, board_j, wi_hbm, wj_hbm, b_hbm, out_hbm,
              vi, vj, idx_i, idx_j, bi, bj, out_v, sem):
    wid = lax.axis_index("s") * NC + lax.axis_index("c")
    bpw = B // NW
    base = wid * bpw

    cps = [
        pltpu.async_copy(board_i.at[pl.ds(base, bpw), pl.ds(0, D)], vi, sem),
        pltpu.async_copy(board_j.at[pl.ds(base, bpw), pl.ds(0, D)], vj, sem),
    ]
    pltpu.sync_copy(wi_hbm.at[pl.ds(base, bpw)], idx_i)
    pltpu.sync_copy(wj_hbm.at[pl.ds(base, bpw)], idx_j)
    bias_cps = []
    for c in range(bpw // 128):
        sl = pl.ds(c * 128, 128)
        bias_cps.append(pltpu.async_copy(b_hbm.at[idx_i.at[sl]], bi.at[sl], sem))
        bias_cps.append(pltpu.async_copy(b_hbm.at[idx_j.at[sl]], bj.at[sl], sem))
    for cp in cps:
        cp.wait()

    lane = _lane()

    def group_body(g, carry):
        rows = g * L + lane
        col = jnp.bitwise_and(lane, D - 1)
        acc = plsc.load_gather(vi, [rows, col]) * plsc.load_gather(vj, [rows, col])
        for k in range(1, D):
            col = jnp.bitwise_and(lane + k, D - 1)
            a = plsc.load_gather(vi, [rows, col])
            bvec = plsc.load_gather(vj, [rows, col])
            acc = acc + a * bvec
        out_v[pl.ds(g * L, L)] = acc
        return carry

    lax.fori_loop(0, bpw // L, group_body, 0)
    for cp in bias_cps:
        cp.wait()

    def bias_body(g, carry):
        sl = pl.ds(g * L, L)
        out_v[sl] = out_v[sl] + bi[sl] + bj[sl]
        return carry

    lax.fori_loop(0, bpw // L, bias_body, 0)
    pltpu.sync_copy(out_v, out_hbm.at[pl.ds(base, bpw)])


@jax.jit
def _glove(w_i, w_j, W, b):
    mesh = plsc.VectorSubcoreMesh(core_axis_name="c", subcore_axis_name="s")
    wt = W.T
    wext = jnp.pad(wt[:, EXTRA_W0:], ((0, 0), (0, 640 - (1000000 - EXTRA_W0))))

    scan = functools.partial(
        pl.kernel,
        mesh=mesh,
        compiler_params=pltpu.CompilerParams(
            needs_layout_passes=False, use_tc_tiling_on_sc=True),
        out_type=(
            pltpu.HBM((PACK * D,), jnp.float32),
            pltpu.HBM((PACK * D,), jnp.float32),
            pltpu.HBM((PACK,), jnp.int32),
            pltpu.HBM((PACK,), jnp.int32),
        ),
        scratch_types=[
            pltpu.VMEM((1024,), jnp.int32),
            pltpu.VMEM((1024,), jnp.int32),
            pltpu.VMEM((CAP,), jnp.int32),
            pltpu.VMEM((CAP,), jnp.int32),
            pltpu.VMEM((CAP,), jnp.int32),
            pltpu.VMEM((CAP,), jnp.int32),
            pltpu.VMEM((CAP,), jnp.int32),
            pltpu.VMEM((CAP,), jnp.int32),
            pltpu.VMEM((CAP,), jnp.int32),
            pltpu.VMEM((CAP,), jnp.int32),
            pltpu.VMEM((CAP * D,), jnp.float32),
            pltpu.VMEM((CAP * D,), jnp.float32),
            pltpu.VMEM((8, CHUNK_W), jnp.float32),
            pltpu.VMEM((8, CHUNK_W), jnp.float32),
            pltpu.SMEM((2 * (NCHUNK + 1),), jnp.int32),
            pltpu.SemaphoreType.DMA,
            pltpu.SemaphoreType.DMA,
        ],
    )(_scan_body)
    packed_i, packed_j, slots_i, slots_j = scan(w_i, w_j, wt, wext)

    board = functools.partial(
        pl.kernel,
        mesh=mesh,
        compiler_params=pltpu.CompilerParams(
            needs_layout_passes=False, use_tc_tiling_on_sc=False),
        out_type=(
            pltpu.HBM((B + PACK, D), jnp.float32),
            pltpu.HBM((B + PACK, D), jnp.float32),
        ),
        scratch_types=[
            pltpu.VMEM((CAP, D), jnp.float32),
            pltpu.VMEM((CAP // 128, 128), jnp.int32),
            pltpu.SemaphoreType.DMA,
        ],
    )(_board_body)
    board_i, board_j = board(
        packed_i.reshape(PACK, D), packed_j.reshape(PACK, D),
        slots_i.reshape(NW, CAP // 128, 128),
        slots_j.reshape(NW, CAP // 128, 128))

    dot = functools.partial(
        pl.kernel,
        mesh=mesh,
        compiler_params=pltpu.CompilerParams(
            needs_layout_passes=False, use_tc_tiling_on_sc=False),
        out_type=jax.ShapeDtypeStruct((B,), jnp.float32),
        scratch_types=[
            pltpu.VMEM((B // NW, D), jnp.float32),
            pltpu.VMEM((B // NW, D), jnp.float32),
            pltpu.VMEM((B // NW,), jnp.int32),
            pltpu.VMEM((B // NW,), jnp.int32),
            pltpu.VMEM((B // NW,), jnp.float32),
            pltpu.VMEM((B // NW,), jnp.float32),
            pltpu.VMEM((B // NW,), jnp.float32),
            pltpu.SemaphoreType.DMA,
        ],
    )(_dot_body)
    return dot(board_i, board_j, w_i, w_j, b)


def kernel(w_i, w_j, W, b):
    return _glove(w_i.astype(jnp.int32), w_j.astype(jnp.int32), W, b)
